# baseline (speedup 1.0000x reference)
import functools
import jax
import jax.numpy as jnp
from jax import lax
from jax.experimental import pallas as pl
from jax.experimental.pallas import tpu as pltpu
from jax.experimental.pallas import tpu_sc as plsc

_NC = 2
_NS = 16
_NW = _NC * _NS
_C = 8


def kernel(x, pos_table):
    batch, seq, d = x.shape
    per_w = seq // _NW
    n_chunks = per_w // _C
    mesh = plsc.VectorSubcoreMesh(core_axis_name="c", subcore_axis_name="s")

    @functools.partial(
        pl.kernel,
        out_type=jax.ShapeDtypeStruct((batch, seq, d), jnp.float32),
        mesh=mesh,
        scratch_types=[
            pltpu.VMEM((2, _C, d), jnp.float32),
            pltpu.VMEM((2, batch, _C, d), jnp.float32),
            pltpu.SemaphoreType.DMA,
            pltpu.SemaphoreType.DMA,
        ],
        compiler_params=pltpu.CompilerParams(use_tc_tiling_on_sc=True),
    )
    def sc_add(x_hbm, pos_hbm, out_hbm, pos_v, x_v, sem_in, sem_out):
        wid = lax.axis_index("s") * _NC + lax.axis_index("c")
        base = wid * per_w

        def issue_in(ci, par):
            s0 = base + ci * _C
            pltpu.async_copy(pos_hbm.at[pl.ds(s0, _C), :], pos_v.at[par], sem_in)
            pltpu.async_copy(x_hbm.at[:, pl.ds(s0, _C), :], x_v.at[par], sem_in)

        def wait_in(par):
            pltpu.make_async_copy(
                pos_hbm.at[pl.ds(0, _C), :], pos_v.at[par], sem_in
            ).wait()
            pltpu.make_async_copy(
                x_hbm.at[:, pl.ds(0, _C), :], x_v.at[par], sem_in
            ).wait()

        def issue_out(ci, par):
            s0 = base + ci * _C
            pltpu.async_copy(x_v.at[par], out_hbm.at[:, pl.ds(s0, _C), :], sem_out)

        def wait_out(par):
            pltpu.make_async_copy(
                x_v.at[par], out_hbm.at[:, pl.ds(0, _C), :], sem_out
            ).wait()

        issue_in(0, 0)

        @pl.loop(0, n_chunks, step=2)
        def _chunks(ci0):
            for par in range(2):
                ci = ci0 + par
                opp = 1 - par

                if par == 0:
                    @pl.when(ci0 >= 1)
                    def _():
                        wait_out(opp)

                    issue_in(ci + 1, opp)
                else:
                    wait_out(opp)

                    @pl.when(ci0 < n_chunks - 2)
                    def _():
                        issue_in(ci + 1, opp)

                wait_in(par)

                for s in range(_C):
                    def _add(i, carry, s=s):
                        sl = pl.ds(i * 16, 16)
                        pv = pos_v[par, s, sl]
                        for b in range(batch):
                            x_v[par, b, s, sl] = x_v[par, b, s, sl] + pv
                        return carry

                    lax.fori_loop(0, d // 16, _add, 0, unroll=4)
                issue_out(ci, par)

        wait_out(1)

    return sc_add(x, pos_table)


# speedup vs baseline: 3.0025x; 3.0025x over previous
import functools
import jax
import jax.numpy as jnp
from jax import lax
from jax.experimental import pallas as pl
from jax.experimental.pallas import tpu as pltpu
from jax.experimental.pallas import tpu_sc as plsc

_NC = 2
_NS = 16
_NW = _NC * _NS
_C = 8


def kernel(x, pos_table):
    batch, seq, d = x.shape
    per_w = seq // _NW
    n_chunks = per_w // _C
    mesh = plsc.VectorSubcoreMesh(core_axis_name="c", subcore_axis_name="s")

    @functools.partial(
        pl.kernel,
        out_type=jax.ShapeDtypeStruct((batch, seq, d), jnp.float32),
        mesh=mesh,
        scratch_types=[
            pltpu.VMEM((2, _C, d), jnp.float32),
            pltpu.VMEM((2, batch, _C, d), jnp.float32),
            pltpu.SemaphoreType.DMA,
            pltpu.SemaphoreType.DMA,
        ],
        compiler_params=pltpu.CompilerParams(use_tc_tiling_on_sc=True),
    )
    def sc_add(x_hbm, pos_hbm, out_hbm, pos_v, x_v, sem_in, sem_out):
        wid = lax.axis_index("s") * _NC + lax.axis_index("c")
        base = wid * per_w

        def issue_in(ci, par):
            s0 = base + ci * _C
            pltpu.async_copy(pos_hbm.at[pl.ds(s0, _C), :], pos_v.at[par], sem_in)
            pltpu.async_copy(x_hbm.at[:, pl.ds(s0, _C), :], x_v.at[par], sem_in)

        def wait_in(par):
            pltpu.make_async_copy(
                pos_hbm.at[pl.ds(0, _C), :], pos_v.at[par], sem_in
            ).wait()
            pltpu.make_async_copy(
                x_hbm.at[:, pl.ds(0, _C), :], x_v.at[par], sem_in
            ).wait()

        def issue_out(ci, par):
            s0 = base + ci * _C
            pltpu.async_copy(x_v.at[par], out_hbm.at[:, pl.ds(s0, _C), :], sem_out)

        def wait_out(par):
            pltpu.make_async_copy(
                x_v.at[par], out_hbm.at[:, pl.ds(0, _C), :], sem_out
            ).wait()

        issue_in(0, 0)

        @pl.loop(0, n_chunks, step=2)
        def _chunks(ci0):
            for par in range(2):
                ci = ci0 + par
                opp = 1 - par

                if par == 0:
                    @pl.when(ci0 >= 1)
                    def _():
                        wait_out(opp)

                    issue_in(ci + 1, opp)
                else:
                    wait_out(opp)

                    @pl.when(ci0 < n_chunks - 2)
                    def _():
                        issue_in(ci + 1, opp)

                wait_in(par)

                for s in range(_C):
                    def _add(i, carry, s=s):
                        sl = pl.ds(i * 16, 16)
                        pv = pos_v[par, s, sl]
                        for b in range(batch):
                            x_v[par, b, s, sl] = x_v[par, b, s, sl] + pv
                        return carry

                    lax.fori_loop(0, d // 16, _add, 0, unroll=8)
                issue_out(ci, par)

        wait_out(1)

    return sc_add(x, pos_table)
